# baseline (speedup 1.0000x reference)
import functools

import jax
import jax.numpy as jnp
from jax import lax
from jax.experimental import pallas as pl
from jax.experimental.pallas import tpu as pltpu
from jax.experimental.pallas import tpu_sc as plsc

N = 90
M = 64 * 28 * 28
NW = 32
CH = 112
NCH = 14
L = 16
INV_S = 1.0 / 28.0

LN2 = 0.6931471805599453
_LOG_COEFFS = (
    7.0376836292e-2, -1.1514610310e-1, 1.1676998740e-1, -1.2420140846e-1,
    1.4249322787e-1, -1.6668057665e-1, 2.0000714765e-1, -2.4999993993e-1,
    3.3333331174e-1,
)


def _plog(x):
    bits = lax.bitcast_convert_type(x, jnp.int32)
    e = ((bits >> 23) & 0xFF) - 126
    m = lax.bitcast_convert_type((bits & 0x007FFFFF) | 0x3F000000, jnp.float32)
    small = m < 0.7071067811865476
    m = jnp.where(small, m + m, m)
    e = jnp.where(small, e - 1, e)
    y = m - 1.0
    z = y * y
    r = jnp.zeros(x.shape, jnp.float32) + _LOG_COEFFS[0]
    for c in _LOG_COEFFS[1:]:
        r = r * y + c
    r = y * z * r - 0.5 * z + y
    return r + e.astype(jnp.float32) * LN2


def _psqrt(x):
    return jnp.exp(0.5 * _plog(x))


def _sc_body(pred_hbm, tgt_hbm, out_hbm,
             p0, t0, p1, t1, idx_v, acc_v, res_v, sem0, sem1):
    c = lax.axis_index("c")
    s = lax.axis_index("s")
    wid = s * 2 + c
    lane = lax.iota(jnp.int32, L)

    acc_v[...] = jnp.zeros((L,), jnp.float32)

    def src_p(i):
        return pred_hbm.at[wid * 2 + i // 7, pl.ds((i % 7) * 4, 4)]

    def src_t(i):
        return tgt_hbm.at[wid * 2 + i // 7, pl.ds((i % 7) * 4, 4)]

    def start(i, pv, tv, sem):
        pltpu.async_copy(src_p(i), pv, sem)
        pltpu.async_copy(src_t(i), tv, sem)

    def wait(i, pv, tv, sem):
        pltpu.make_async_copy(src_p(i), pv, sem).wait()
        pltpu.make_async_copy(src_t(i), tv, sem).wait()

    def col(f):
        return jnp.zeros((L,), jnp.int32) + f

    def ijsplit(r):
        iv = (r * 9363) >> 18
        return iv, r - iv * 28

    def compute(pred_v, tgt_v):
        def pass1(g, carry):
            acc1, k_vec = carry
            lr = lane + g * L
            iv, jv = ijsplit(lr)
            c4 = col(4)
            confv = plsc.load_gather(tgt_v, [iv, jv, c4])
            qp = plsc.load_gather(pred_v, [iv, jv, c4])
            alpha = (1.0 - confv) / (1.0 - qp)
            df = (alpha * (qp - confv) * _plog(qp)
                  + (confv - qp) * _plog(1.0 - qp))
            acc1 = acc1 + df
            m = confv > 0.0
            pos = jnp.cumsum(m.astype(jnp.int32))
            dst = jnp.where(m, k_vec + pos - 1, CH + 15)
            plsc.store_scatter(idx_v, [dst], lr, mask=m)
            k_vec = k_vec + plsc.all_reduce_population_count(m)
            return acc1, k_vec

        acc, k_vec = lax.fori_loop(
            0, CH // L, pass1,
            (jnp.zeros((L,), jnp.float32), jnp.zeros((L,), jnp.int32)))

        k_s = jnp.max(k_vec)
        ngroups = (k_s + (L - 1)) // L

        def mgroup(g, acc2):
            pos16 = lane + g * L
            valid = pos16 < k_vec
            r = plsc.load_gather(idx_v, [pos16])
            r = jnp.where(valid, r, 0)
            iv, jv = ijsplit(r)

            def pg(f):
                return plsc.load_gather(pred_v, [iv, jv, col(f)])

            def tg(f):
                return plsc.load_gather(tgt_v, [iv, jv, col(f)])

            px0, py0, pw0, ph0, pc0 = pg(0), pg(1), pg(2), pg(3), pg(4)
            px1, py1, pw1, ph1, pc1 = pg(5), pg(6), pg(7), pg(8), pg(9)
            tx, ty, tw, th = tg(0), tg(1), tg(2), tg(3)
            tx1, ty1, tw1, th1 = tg(5), tg(6), tg(7), tg(8)

            tltx = tx * INV_S - 0.5 * tw
            trbx = tx * INV_S + 0.5 * tw
            tlty = ty * INV_S - 0.5 * th
            trby = ty * INV_S + 0.5 * th
            area_t = (trbx - tltx) * (trby - tlty)

            def iou(px, py, pw, ph):
                pltx = px * INV_S - 0.5 * pw
                prbx = px * INV_S + 0.5 * pw
                plty = py * INV_S - 0.5 * ph
                prby = py * INV_S + 0.5 * ph
                whx = jnp.maximum(
                    jnp.minimum(prbx, trbx) - jnp.maximum(pltx, tltx), 0.0)
                why = jnp.maximum(
                    jnp.minimum(prby, trby) - jnp.maximum(plty, tlty), 0.0)
                inter = whx * why
                area_p = (prbx - pltx) * (prby - plty)
                return inter / (area_p + area_t - inter)

            i0 = iou(px0, py0, pw0, ph0)
            i1 = iou(px1, py1, pw1, ph1)
            selb = i1 > i0
            mx = jnp.maximum(i0, i1)

            def pick(a, b):
                return jnp.where(selb, b, a)

            spx, spy = pick(px0, px1), pick(py0, py1)
            spw, sph, spc = pick(pw0, pw1), pick(ph0, ph1), pick(pc0, pc1)
            stx, sty = pick(tx, tx1), pick(ty, ty1)
            stw, sth = pick(tw, tw1), pick(th, th1)

            dx, dy, dc = spx - stx, spy - sty, spc - mx
            tot = dx * dx + dy * dy + dc * dc
            tot = tot + spw + stw - 2.0 * _psqrt(spw * stw)
            tot = tot + sph + sth - 2.0 * _psqrt(sph * sth)

            def cls_chunk(j, bce):
                cb = 10 + j * L
                for u in range(L):
                    fc = col(cb + u)
                    pcv = plsc.load_gather(pred_v, [iv, jv, fc])
                    tcv = plsc.load_gather(tgt_v, [iv, jv, fc])
                    bce = bce - (tcv * _plog(pcv)
                                 + (1.0 - tcv) * _plog(1.0 - pcv))
                return bce

            bce = lax.fori_loop(0, (N - 10) // L, cls_chunk,
                                jnp.zeros((L,), jnp.float32))
            tot = tot + bce
            return acc2 + jnp.where(valid, tot, 0.0)

        acc = lax.fori_loop(0, ngroups, mgroup, acc)
        acc_v[...] = acc_v[...] + acc

    start(0, p0, t0, sem0)

    def body(i, carry):
        @pl.when((i & 1) == 0)
        def _():
            @pl.when(i + 1 < NCH)
            def _():
                start(i + 1, p1, t1, sem1)
            wait(i, p0, t0, sem0)
            compute(p0, t0)

        @pl.when((i & 1) == 1)
        def _():
            @pl.when(i + 1 < NCH)
            def _():
                start(i + 1, p0, t0, sem0)
            wait(i, p1, t1, sem1)
            compute(p1, t1)

        return carry

    lax.fori_loop(0, NCH, body, 0)
    res_v[...] = acc_v[...]
    pltpu.sync_copy(res_v, out_hbm.at[wid])


_MESH = plsc.VectorSubcoreMesh(core_axis_name="c", subcore_axis_name="s")

_sc_call = functools.partial(
    pl.kernel,
    out_type=jax.ShapeDtypeStruct((NW, L), jnp.float32),
    mesh=_MESH,
    compiler_params=pltpu.CompilerParams(
        needs_layout_passes=False, use_tc_tiling_on_sc=False),
    scratch_types=[
        pltpu.VMEM((4, 28, N), jnp.float32),
        pltpu.VMEM((4, 28, N), jnp.float32),
        pltpu.VMEM((4, 28, N), jnp.float32),
        pltpu.VMEM((4, 28, N), jnp.float32),
        pltpu.VMEM((CH + L,), jnp.int32),
        pltpu.VMEM((L,), jnp.float32),
        pltpu.VMEM((L,), jnp.float32),
        pltpu.SemaphoreType.DMA,
        pltpu.SemaphoreType.DMA,
    ],
)(_sc_body)


def kernel(pred_tensor, target_tensor):
    parts = _sc_call(pred_tensor, target_tensor)
    return jnp.sum(parts)


# speedup vs baseline: 1.5583x; 1.5583x over previous
import functools

import jax
import jax.numpy as jnp
from jax import lax
from jax.experimental import pallas as pl
from jax.experimental.pallas import tpu as pltpu
from jax.experimental.pallas import tpu_sc as plsc

N = 90
M = 64 * 28 * 28
NW = 32
CH = 224
NCH = M // CH // NW
CW = CH * N
L = 16
INV_S = 1.0 / 28.0

LN2 = 0.6931471805599453
_LOG_COEFFS = (
    7.0376836292e-2, -1.1514610310e-1, 1.1676998740e-1, -1.2420140846e-1,
    1.4249322787e-1, -1.6668057665e-1, 2.0000714765e-1, -2.4999993993e-1,
    3.3333331174e-1,
)


def _plog(x):
    bits = lax.bitcast_convert_type(x, jnp.int32)
    e = ((bits >> 23) & 0xFF) - 126
    m = lax.bitcast_convert_type((bits & 0x007FFFFF) | 0x3F000000, jnp.float32)
    small = m < 0.7071067811865476
    m = jnp.where(small, m + m, m)
    e = jnp.where(small, e - 1, e)
    y = m - 1.0
    z = y * y
    r = jnp.zeros(x.shape, jnp.float32) + _LOG_COEFFS[0]
    for c in _LOG_COEFFS[1:]:
        r = r * y + c
    r = y * z * r - 0.5 * z + y
    return r + e.astype(jnp.float32) * LN2


def _psqrt(x):
    return jnp.exp(0.5 * _plog(x))


def _sc_body(pred_hbm, tgt_hbm, out_hbm,
             p0, t0, p1, t1, idx_v, acc_v, res_v, sem0, sem1):
    c = lax.axis_index("c")
    s = lax.axis_index("s")
    wid = s * 2 + c
    lane = lax.iota(jnp.int32, L)
    acc_v[...] = jnp.zeros((L,), jnp.float32)

    def src_p(i):
        return pred_hbm.at[pl.ds((wid + i * NW) * CW, CW)]

    def src_t(i):
        return tgt_hbm.at[pl.ds((wid + i * NW) * CW, CW)]

    def start(i, pv, tv, sem):
        pltpu.async_copy(src_p(i), pv, sem)
        pltpu.async_copy(src_t(i), tv, sem)

    def wait(i, pv, tv, sem):
        pltpu.make_async_copy(src_p(i), pv, sem).wait()
        pltpu.make_async_copy(src_t(i), tv, sem).wait()

    def gat(ref, w):
        return plsc.load_gather(ref, [w])

    def compute(pred_v, tgt_v):
        def pass1(g, carry):
            acc1, k_vec = carry
            lr = lane + g * L
            w4 = lr * N + 4
            confv = gat(tgt_v, w4)
            qp = gat(pred_v, w4)
            alpha = (1.0 - confv) / (1.0 - qp)
            df = (alpha * (qp - confv) * _plog(qp)
                  + (confv - qp) * _plog(1.0 - qp))
            acc1 = acc1 + df
            m = confv > 0.0
            pos = jnp.cumsum(m.astype(jnp.int32))
            dst = jnp.where(m, k_vec + pos - 1, CH + 15)
            plsc.store_scatter(idx_v, [dst], lr, mask=m)
            k_vec = k_vec + plsc.all_reduce_population_count(m)
            return acc1, k_vec

        acc, k_vec = lax.fori_loop(
            0, CH // L, pass1,
            (jnp.zeros((L,), jnp.float32), jnp.zeros((L,), jnp.int32)))

        k_s = jnp.max(k_vec)
        ngroups = (k_s + (L - 1)) // L

        def mgroup(g, acc2):
            pos16 = lane + g * L
            valid = pos16 < k_vec
            r = plsc.load_gather(idx_v, [pos16])
            r = jnp.where(valid, r, 0)
            rw = r * N

            def pg(f):
                return gat(pred_v, rw + f)

            def tg(f):
                return gat(tgt_v, rw + f)

            px0, py0, pw0, ph0, pc0 = pg(0), pg(1), pg(2), pg(3), pg(4)
            px1, py1, pw1, ph1, pc1 = pg(5), pg(6), pg(7), pg(8), pg(9)
            tx, ty, tw, th = tg(0), tg(1), tg(2), tg(3)
            tx1, ty1, tw1, th1 = tg(5), tg(6), tg(7), tg(8)

            tltx = tx * INV_S - 0.5 * tw
            trbx = tx * INV_S + 0.5 * tw
            tlty = ty * INV_S - 0.5 * th
            trby = ty * INV_S + 0.5 * th
            area_t = (trbx - tltx) * (trby - tlty)

            def iou(px, py, pw, ph):
                pltx = px * INV_S - 0.5 * pw
                prbx = px * INV_S + 0.5 * pw
                plty = py * INV_S - 0.5 * ph
                prby = py * INV_S + 0.5 * ph
                whx = jnp.maximum(
                    jnp.minimum(prbx, trbx) - jnp.maximum(pltx, tltx), 0.0)
                why = jnp.maximum(
                    jnp.minimum(prby, trby) - jnp.maximum(plty, tlty), 0.0)
                inter = whx * why
                area_p = (prbx - pltx) * (prby - plty)
                return inter / (area_p + area_t - inter)

            i0 = iou(px0, py0, pw0, ph0)
            i1 = iou(px1, py1, pw1, ph1)
            selb = i1 > i0
            mx = jnp.maximum(i0, i1)

            def pick(a, b):
                return jnp.where(selb, b, a)

            spx, spy = pick(px0, px1), pick(py0, py1)
            spw, sph, spc = pick(pw0, pw1), pick(ph0, ph1), pick(pc0, pc1)
            stx, sty = pick(tx, tx1), pick(ty, ty1)
            stw, sth = pick(tw, tw1), pick(th, th1)

            dx, dy, dc = spx - stx, spy - sty, spc - mx
            tot = dx * dx + dy * dy + dc * dc
            tot = tot + spw + stw - 2.0 * _psqrt(spw * stw)
            tot = tot + sph + sth - 2.0 * _psqrt(sph * sth)

            def cls_chunk(j, bce):
                cb = 10 + j * L
                for u in range(L):
                    wv = rw + (cb + u)
                    pcv = gat(pred_v, wv)
                    tcv = gat(tgt_v, wv)
                    bce = bce - (tcv * _plog(pcv)
                                 + (1.0 - tcv) * _plog(1.0 - pcv))
                return bce

            bce = lax.fori_loop(0, (N - 10) // L, cls_chunk,
                                jnp.zeros((L,), jnp.float32))
            tot = tot + bce
            return acc2 + jnp.where(valid, tot, 0.0)

        acc = lax.fori_loop(0, ngroups, mgroup, acc)
        acc_v[...] = acc_v[...] + acc

    start(0, p0, t0, sem0)

    def body(i, carry):
        @pl.when((i & 1) == 0)
        def _():
            @pl.when(i + 1 < NCH)
            def _():
                start(i + 1, p1, t1, sem1)
            wait(i, p0, t0, sem0)
            compute(p0, t0)

        @pl.when((i & 1) == 1)
        def _():
            @pl.when(i + 1 < NCH)
            def _():
                start(i + 1, p0, t0, sem0)
            wait(i, p1, t1, sem1)
            compute(p1, t1)

        return carry

    lax.fori_loop(0, NCH, body, 0)
    res_v[...] = acc_v[...]
    pltpu.sync_copy(res_v, out_hbm.at[wid])


_MESH = plsc.VectorSubcoreMesh(core_axis_name="c", subcore_axis_name="s")

_sc_call = functools.partial(
    pl.kernel,
    out_type=jax.ShapeDtypeStruct((NW, L), jnp.float32),
    mesh=_MESH,
    compiler_params=pltpu.CompilerParams(needs_layout_passes=False),
    scratch_types=[
        pltpu.VMEM((CW,), jnp.float32),
        pltpu.VMEM((CW,), jnp.float32),
        pltpu.VMEM((CW,), jnp.float32),
        pltpu.VMEM((CW,), jnp.float32),
        pltpu.VMEM((CH + L,), jnp.int32),
        pltpu.VMEM((L,), jnp.float32),
        pltpu.VMEM((L,), jnp.float32),
        pltpu.SemaphoreType.DMA,
        pltpu.SemaphoreType.DMA,
    ],
)(_sc_body)


def kernel(pred_tensor, target_tensor):
    p = pred_tensor.reshape(M * N)
    t = target_tensor.reshape(M * N)
    parts = _sc_call(p, t)
    return jnp.sum(parts)
